# Initial kernel scaffold; baseline (speedup 1.0000x reference)
import jax
import jax.numpy as jnp
from jax.experimental import pallas as pl


def kernel(x, protos):
    raise NotImplementedError("write your pallas kernel here")


# speedup vs baseline: 12.6271x; 12.6271x over previous
import jax
import jax.numpy as jnp
from jax.experimental import pallas as pl
from jax.experimental.pallas import tpu as pltpu

_NCLS = 512


def _glvq_body(x_ref, p_ref, o_ref):
    x = x_ref[:]
    pa = p_ref[:_NCLS, :]
    pb = p_ref[_NCLS:, :]
    xx = jnp.sum(x * x, axis=1, keepdims=True)
    dn = (((1,), (1,)), ((), ()))
    xa = jax.lax.dot_general(x, pa, dn, preferred_element_type=jnp.float32)
    xb = jax.lax.dot_general(x, pb, dn, preferred_element_type=jnp.float32)
    da = xx - 2.0 * xa + jnp.sum(pa * pa, axis=1)[None, :]
    db = xx - 2.0 * xb + jnp.sum(pb * pb, axis=1)[None, :]
    o_ref[:] = jnp.sqrt(jnp.maximum(jnp.minimum(da, db), 0.0))


def kernel(x, protos):
    batch = x.shape[0]
    return pl.pallas_call(
        _glvq_body,
        out_shape=jax.ShapeDtypeStruct((batch, _NCLS), jnp.float32),
    )(x, protos)
